# Initial kernel scaffold; baseline (speedup 1.0000x reference)
import jax
import jax.numpy as jnp
from jax.experimental import pallas as pl


def kernel(x, params):
    raise NotImplementedError("write your pallas kernel here")


# speedup vs baseline: 16.7941x; 16.7941x over previous
import functools

import jax
import jax.numpy as jnp
from jax.experimental import pallas as pl
from jax.experimental.pallas import tpu as pltpu

D_MODEL = 1024
NUM_EXPERTS = 8
N_CONV = 4
N_MAMBA = 4
TOP_K = 2
KCONV = 7
D_STATE = 16
D_CONV = 4
D_HID = 2048
DT_RANK = 64
T = 2048
WIN = 64

F32 = jnp.float32


def _router_kernel(x_ref, nw_ref, gw_ref, ew_ref, eb_ref, temp_ref,
                   xn_ref, wmask_ref, aux_ref):
    x = x_ref[...]
    xn = x * jax.lax.rsqrt(jnp.mean(x * x, axis=1, keepdims=True) + 1e-6)
    xn = xn * nw_ref[...]
    xn_ref[...] = xn

    xm = jnp.mean(xn, axis=1, keepdims=True)
    s = jnp.concatenate([xm, xm * xm], axis=1)
    for k in range(6):
        sh = 1 << k
        s = s + jnp.concatenate([jnp.zeros((sh, 2), F32), s[:T - sh]], axis=0)
    mu = s[:, 0:1] * (1.0 / WIN)
    mu_sq = s[:, 1:2] * (1.0 / WIN)
    var = jnp.maximum(mu_sq - mu * mu, 0.0)
    ent = (jnp.log(var + 1e-6) + 10.0) / 20.0

    logits = jax.lax.dot_general(xn, gw_ref[...], (((1,), (0,)), ((), ())),
                                 preferred_element_type=F32)
    logits = logits + ent * ew_ref[...] + eb_ref[...]
    logits = logits / (jnp.abs(temp_ref[0, 0]) + 1e-6)

    idx = jax.lax.broadcasted_iota(jnp.int32, (1, NUM_EXPERTS), 1)
    m1 = jnp.max(logits, axis=1, keepdims=True)
    a1 = jnp.min(jnp.where(logits == m1, idx, NUM_EXPERTS), axis=1,
                 keepdims=True)
    masked = jnp.where(idx == a1, -1e30, logits)
    m2 = jnp.max(masked, axis=1, keepdims=True)
    a2 = jnp.min(jnp.where(masked == m2, idx, NUM_EXPERTS), axis=1,
                 keepdims=True)
    w1 = 1.0 / (1.0 + jnp.exp(m2 - m1))
    w2 = 1.0 - w1
    sel1 = (idx == a1).astype(F32)
    sel2 = (idx == a2).astype(F32)
    wmask_ref[...] = w1 * sel1 + w2 * sel2

    p = jnp.exp(logits - m1)
    probs = p / jnp.sum(p, axis=1, keepdims=True)
    avg_probs = jnp.mean(probs, axis=0, keepdims=True)
    tpe = jnp.mean(sel1 + sel2, axis=0, keepdims=True)
    aux_ref[...] = NUM_EXPERTS * jnp.sum(tpe * avg_probs, axis=1,
                                         keepdims=True)


def _run_router(xs, p):
    return pl.pallas_call(
        _router_kernel,
        out_shape=(jax.ShapeDtypeStruct((T, D_MODEL), F32),
                   jax.ShapeDtypeStruct((T, NUM_EXPERTS), F32),
                   jax.ShapeDtypeStruct((1, 1), F32)),
    )(xs, p['norm_w'].reshape(1, D_MODEL), p['gate_w'],
      p['entropy_w'], p['entropy_b'].reshape(1, NUM_EXPERTS),
      p['temperature'].reshape(1, 1))


_CJ = 512
_NCJ = D_HID // _CJ


def _gelu_exact(v):
    return 0.5 * v * (1.0 + jax.lax.erf(v * 0.7071067811865476))


def _shift_down(h, s, width):
    if s == 0:
        return h
    return jnp.concatenate([jnp.zeros((s, width), F32), h[:T - s]], axis=0)


def _conv_experts_kernel(xn_ref, x_ref, wmask_ref, win_ref, bin_ref, kw_ref,
                         kb_ref, wout_ref, bout_ref, out_ref):
    e = pl.program_id(0)
    j = pl.program_id(1)
    xn = xn_ref[...]
    h = jax.lax.dot_general(xn, win_ref[0], (((1,), (0,)), ((), ())),
                            preferred_element_type=F32) + bin_ref[0]
    h = _gelu_exact(h)
    hc = kb_ref[0]
    for tap in range(KCONV):
        hc = hc + kw_ref[0, tap, :][None, :] * _shift_down(h, KCONV - 1 - tap,
                                                           _CJ)
    h2 = _gelu_exact(hc)
    part = jax.lax.dot_general(h2, wout_ref[0], (((1,), (0,)), ((), ())),
                               preferred_element_type=F32)
    eidx = jax.lax.broadcasted_iota(jnp.int32, (1, NUM_EXPERTS), 1)
    wm = jnp.sum(wmask_ref[...] * (eidx == e).astype(F32), axis=1,
                 keepdims=True)
    contrib = part * wm
    contrib = jnp.where(j == _NCJ - 1, contrib + bout_ref[0] * wm, contrib)

    @pl.when((e == 0) & (j == 0))
    def _init():
        out_ref[...] = x_ref[...] + contrib

    @pl.when(jnp.logical_not((e == 0) & (j == 0)))
    def _acc():
        out_ref[...] = out_ref[...] + contrib


def _run_conv_experts(xn, xs, wmask, cw):
    grid = (N_CONV, _NCJ)
    return pl.pallas_call(
        _conv_experts_kernel,
        grid=grid,
        in_specs=[
            pl.BlockSpec((T, D_MODEL), lambda e, j: (0, 0)),
            pl.BlockSpec((T, D_MODEL), lambda e, j: (0, 0)),
            pl.BlockSpec((T, NUM_EXPERTS), lambda e, j: (0, 0)),
            pl.BlockSpec((1, D_MODEL, _CJ), lambda e, j: (e, 0, j)),
            pl.BlockSpec((1, 1, _CJ), lambda e, j: (e, 0, j)),
            pl.BlockSpec((1, 8, _CJ), lambda e, j: (e, 0, j)),
            pl.BlockSpec((1, 1, _CJ), lambda e, j: (e, 0, j)),
            pl.BlockSpec((1, _CJ, D_MODEL), lambda e, j: (e, j, 0)),
            pl.BlockSpec((1, 1, D_MODEL), lambda e, j: (e, 0, 0)),
        ],
        out_specs=pl.BlockSpec((T, D_MODEL), lambda e, j: (0, 0)),
        out_shape=jax.ShapeDtypeStruct((T, D_MODEL), F32),
    )(xn, xs, wmask, cw['win'], cw['bin'], cw['kw'], cw['kb'], cw['wout'],
      cw['bout'])


_MJ = 512
_NMJ = D_HID // _MJ


def _mamba_front_kernel(xn_ref, wxi_ref, wres_ref, kw_ref, kb_ref, wx_ref,
                        xi_ref, res_ref, dbc_ref):
    j = pl.program_id(1)
    xn = xn_ref[...]
    xi = jax.lax.dot_general(xn, wxi_ref[0], (((1,), (0,)), ((), ())),
                             preferred_element_type=F32)
    hc = kb_ref[0]
    for tap in range(D_CONV):
        hc = hc + kw_ref[0, tap, :][None, :] * _shift_down(xi, D_CONV - 1 - tap,
                                                           _MJ)
    xi_s = jax.nn.silu(hc)
    xi_ref[0] = xi_s
    res = jax.lax.dot_general(xn, wres_ref[0], (((1,), (0,)), ((), ())),
                              preferred_element_type=F32)
    res_ref[0] = jax.nn.silu(res)
    part = jax.lax.dot_general(xi_s, wx_ref[0], (((1,), (0,)), ((), ())),
                               preferred_element_type=F32)

    @pl.when(j == 0)
    def _init():
        dbc_ref[0] = part

    @pl.when(j != 0)
    def _acc():
        dbc_ref[0] = dbc_ref[0] + part


def _run_mamba_front(xn, mw):
    grid = (N_MAMBA, _NMJ)
    return pl.pallas_call(
        _mamba_front_kernel,
        grid=grid,
        in_specs=[
            pl.BlockSpec((T, D_MODEL), lambda e, j: (0, 0)),
            pl.BlockSpec((1, D_MODEL, _MJ), lambda e, j: (e, 0, j)),
            pl.BlockSpec((1, D_MODEL, _MJ), lambda e, j: (e, 0, j)),
            pl.BlockSpec((1, 8, _MJ), lambda e, j: (e, 0, j)),
            pl.BlockSpec((1, 1, _MJ), lambda e, j: (e, 0, j)),
            pl.BlockSpec((1, _MJ, 128), lambda e, j: (e, j, 0)),
        ],
        out_specs=[
            pl.BlockSpec((1, T, _MJ), lambda e, j: (e, 0, j)),
            pl.BlockSpec((1, T, _MJ), lambda e, j: (e, 0, j)),
            pl.BlockSpec((1, T, 128), lambda e, j: (e, 0, 0)),
        ],
        out_shape=[
            jax.ShapeDtypeStruct((N_MAMBA, T, D_HID), F32),
            jax.ShapeDtypeStruct((N_MAMBA, T, D_HID), F32),
            jax.ShapeDtypeStruct((N_MAMBA, T, 128), F32),
        ],
    )(xn, mw['wxi'], mw['wres'], mw['kw'], mw['kb'], mw['wx'])


_SJ = 256
_NSJ = D_HID // _SJ
_NC = 64
_CL = T // _NC


def _scan_kernel(xi_ref, dbc_ref, wdt_ref, bdt_ref, at_ref, d_ref,
                 y_ref, ys_scr, srel_scr):
    dbc = dbc_ref[0]
    dt = jax.lax.dot_general(dbc, wdt_ref[0], (((1,), (0,)), ((), ())),
                             preferred_element_type=F32) + bdt_ref[0]
    delta = jnp.maximum(dt, 0.0) + jnp.log1p(jnp.exp(-jnp.abs(dt)))
    delta_r = delta.reshape(_NC, _CL, _SJ)
    xi = xi_ref[0]
    xi_r = xi.reshape(_NC, _CL, _SJ)
    dbc_r = dbc.reshape(_NC, _CL, 128)
    b_all = jnp.swapaxes(dbc_r[:, :, DT_RANK:DT_RANK + D_STATE], 1, 2)
    c_all = jnp.swapaxes(
        dbc_r[:, :, DT_RANK + D_STATE:DT_RANK + 2 * D_STATE], 1, 2)
    at = at_ref[0]

    h = jnp.zeros((_NC, D_STATE, _SJ), F32)
    ssum = jnp.zeros((_NC, 1, _SJ), F32)
    for l in range(_CL):
        d_l = delta_r[:, l:l + 1, :]
        dA = jnp.exp(d_l * at[None, :, :])
        b_l = b_all[:, :, l:l + 1]
        x_l = xi_r[:, l:l + 1, :]
        h = dA * h + (d_l * x_l) * b_l
        c_l = c_all[:, :, l:l + 1]
        ys_scr[:, l, :] = jnp.sum(h * c_l, axis=1)
        ssum = ssum + d_l
        srel_scr[:, l, :] = ssum[:, 0, :]

    P = jnp.exp(at[None, :, :] * ssum)
    H = h
    k = 1
    while k < _NC:
        Hs = jnp.concatenate([jnp.zeros((k, D_STATE, _SJ), F32), H[:_NC - k]],
                             axis=0)
        Ps = jnp.concatenate([jnp.ones((k, D_STATE, _SJ), F32), P[:_NC - k]],
                             axis=0)
        H = H + P * Hs
        P = P * Ps
        k *= 2
    hprev = jnp.concatenate([jnp.zeros((1, D_STATE, _SJ), F32), H[:_NC - 1]],
                            axis=0)

    srel = srel_scr[...]
    corr = jnp.zeros((_NC, _CL, _SJ), F32)
    for n in range(D_STATE):
        c_n = dbc_r[:, :, DT_RANK + D_STATE + n:DT_RANK + D_STATE + n + 1]
        decay = jnp.exp(at[n, :][None, None, :] * srel)
        corr = corr + c_n * decay * hprev[:, n:n + 1, :]
    y = ys_scr[...] + corr
    y_ref[0] = y.reshape(T, _SJ) + xi * d_ref[0]


def _run_scan(xi_s, dbc, mw):
    grid = (N_MAMBA, _NSJ)
    return pl.pallas_call(
        _scan_kernel,
        grid=grid,
        in_specs=[
            pl.BlockSpec((1, T, _SJ), lambda e, j: (e, 0, j)),
            pl.BlockSpec((1, T, 128), lambda e, j: (e, 0, 0)),
            pl.BlockSpec((1, 128, _SJ), lambda e, j: (e, 0, j)),
            pl.BlockSpec((1, 1, _SJ), lambda e, j: (e, 0, j)),
            pl.BlockSpec((1, D_STATE, _SJ), lambda e, j: (e, 0, j)),
            pl.BlockSpec((1, 1, _SJ), lambda e, j: (e, 0, j)),
        ],
        out_specs=pl.BlockSpec((1, T, _SJ), lambda e, j: (e, 0, j)),
        out_shape=jax.ShapeDtypeStruct((N_MAMBA, T, D_HID), F32),
        scratch_shapes=[
            pltpu.VMEM((_NC, _CL, _SJ), F32),
            pltpu.VMEM((_NC, _CL, _SJ), F32),
        ],
    )(xi_s, dbc, mw['wdt'], mw['bdt'], mw['at'], mw['d'])


def _mamba_tail_kernel(y_ref, res_ref, wmask_ref, wout_ref, base_ref, out_ref):
    e = pl.program_id(0)
    j = pl.program_id(1)
    g = y_ref[0] * res_ref[0]
    part = jax.lax.dot_general(g, wout_ref[0], (((1,), (0,)), ((), ())),
                               preferred_element_type=F32)
    eidx = jax.lax.broadcasted_iota(jnp.int32, (1, NUM_EXPERTS), 1)
    wm = jnp.sum(wmask_ref[...] * (eidx == (e + N_CONV)).astype(F32), axis=1,
                 keepdims=True)
    contrib = part * wm

    @pl.when((e == 0) & (j == 0))
    def _init():
        out_ref[...] = base_ref[...] + contrib

    @pl.when(jnp.logical_not((e == 0) & (j == 0)))
    def _acc():
        out_ref[...] = out_ref[...] + contrib


def _run_mamba_tail(y_full, res_s, wmask, base, mw):
    grid = (N_MAMBA, _NMJ)
    return pl.pallas_call(
        _mamba_tail_kernel,
        grid=grid,
        in_specs=[
            pl.BlockSpec((1, T, _MJ), lambda e, j: (e, 0, j)),
            pl.BlockSpec((1, T, _MJ), lambda e, j: (e, 0, j)),
            pl.BlockSpec((T, NUM_EXPERTS), lambda e, j: (0, 0)),
            pl.BlockSpec((1, _MJ, D_MODEL), lambda e, j: (e, j, 0)),
            pl.BlockSpec((T, D_MODEL), lambda e, j: (0, 0)),
        ],
        out_specs=pl.BlockSpec((T, D_MODEL), lambda e, j: (0, 0)),
        out_shape=jax.ShapeDtypeStruct((T, D_MODEL), F32),
    )(y_full, res_s, wmask, mw['wout'], base)


def _stack_weights(params):
    ce = params['conv_experts']
    cw = {
        'win': jnp.stack([p['fc_in_w'] for p in ce]),
        'bin': jnp.stack([p['fc_in_b'].reshape(1, D_HID) for p in ce]),
        'kw': jnp.stack([
            jnp.pad(jnp.transpose(p['conv_w'][:, 0, :]), ((0, 1), (0, 0)))
            for p in ce]),
        'kb': jnp.stack([p['conv_b'].reshape(1, D_HID) for p in ce]),
        'wout': jnp.stack([p['fc_out_w'] for p in ce]),
        'bout': jnp.stack([p['fc_out_b'].reshape(1, D_MODEL) for p in ce]),
    }
    me = params['mamba_experts']
    mw = {
        'wxi': jnp.stack([p['in_proj_w'][:, :D_HID] for p in me]),
        'wres': jnp.stack([p['in_proj_w'][:, D_HID:] for p in me]),
        'kw': jnp.stack([
            jnp.pad(jnp.transpose(p['conv_w'][:, 0, :]), ((0, 4), (0, 0)))
            for p in me]),
        'kb': jnp.stack([p['conv_b'].reshape(1, D_HID) for p in me]),
        'wx': jnp.stack([
            jnp.pad(p['x_proj_w'], ((0, 0), (0, 128 - (DT_RANK + 2 * D_STATE))))
            for p in me]),
        'wdt': jnp.stack([
            jnp.pad(p['dt_proj_w'], ((0, 128 - DT_RANK), (0, 0)))
            for p in me]),
        'bdt': jnp.stack([p['dt_proj_b'].reshape(1, D_HID) for p in me]),
        'at': jnp.stack([jnp.transpose(-jnp.exp(p['A_log'])) for p in me]),
        'd': jnp.stack([p['D'].reshape(1, D_HID) for p in me]),
        'wout': jnp.stack([p['out_proj_w'] for p in me]),
    }
    return cw, mw


@jax.jit
def kernel(x, params):
    xs = x[0]
    cw, mw = _stack_weights(params)
    xn, wmask, aux = _run_router(xs, params)
    base = _run_conv_experts(xn, xs, wmask, cw)
    xi_s, res_s, dbc = _run_mamba_front(xn, mw)
    y_full = _run_scan(xi_s, dbc, mw)
    out = _run_mamba_tail(y_full, res_s, wmask, base, mw)
    return out[None], aux[0, 0]


# speedup vs baseline: 17.1192x; 1.0194x over previous
import functools

import jax
import jax.numpy as jnp
from jax.experimental import pallas as pl
from jax.experimental.pallas import tpu as pltpu

D_MODEL = 1024
NUM_EXPERTS = 8
N_CONV = 4
N_MAMBA = 4
TOP_K = 2
KCONV = 7
D_STATE = 16
D_CONV = 4
D_HID = 2048
DT_RANK = 64
T = 2048
WIN = 64

F32 = jnp.float32
BF16 = jnp.bfloat16


def _mm(a, b):
    return jax.lax.dot_general(a.astype(BF16), b.astype(BF16),
                               (((1,), (0,)), ((), ())),
                               preferred_element_type=F32)


def _router_kernel(x_ref, nw_ref, gw_ref, ew_ref, eb_ref, temp_ref,
                   xn_ref, wmask_ref, aux_ref):
    x = x_ref[...]
    xn = x * jax.lax.rsqrt(jnp.mean(x * x, axis=1, keepdims=True) + 1e-6)
    xn = xn * nw_ref[...]
    xn_ref[...] = xn

    xm = jnp.mean(xn, axis=1, keepdims=True)
    s = jnp.concatenate([xm, xm * xm], axis=1)
    for k in range(6):
        sh = 1 << k
        s = s + jnp.concatenate([jnp.zeros((sh, 2), F32), s[:T - sh]], axis=0)
    mu = s[:, 0:1] * (1.0 / WIN)
    mu_sq = s[:, 1:2] * (1.0 / WIN)
    var = jnp.maximum(mu_sq - mu * mu, 0.0)
    ent = (jnp.log(var + 1e-6) + 10.0) / 20.0

    logits = jax.lax.dot_general(xn, gw_ref[...], (((1,), (0,)), ((), ())),
                                 preferred_element_type=F32)
    logits = logits + ent * ew_ref[...] + eb_ref[...]
    logits = logits / (jnp.abs(temp_ref[0, 0]) + 1e-6)

    idx = jax.lax.broadcasted_iota(jnp.int32, (1, NUM_EXPERTS), 1)
    m1 = jnp.max(logits, axis=1, keepdims=True)
    a1 = jnp.min(jnp.where(logits == m1, idx, NUM_EXPERTS), axis=1,
                 keepdims=True)
    masked = jnp.where(idx == a1, -1e30, logits)
    m2 = jnp.max(masked, axis=1, keepdims=True)
    a2 = jnp.min(jnp.where(masked == m2, idx, NUM_EXPERTS), axis=1,
                 keepdims=True)
    w1 = 1.0 / (1.0 + jnp.exp(m2 - m1))
    w2 = 1.0 - w1
    sel1 = (idx == a1).astype(F32)
    sel2 = (idx == a2).astype(F32)
    wmask_ref[...] = w1 * sel1 + w2 * sel2

    p = jnp.exp(logits - m1)
    probs = p / jnp.sum(p, axis=1, keepdims=True)
    avg_probs = jnp.mean(probs, axis=0, keepdims=True)
    tpe = jnp.mean(sel1 + sel2, axis=0, keepdims=True)
    aux_ref[...] = NUM_EXPERTS * jnp.sum(tpe * avg_probs, axis=1,
                                         keepdims=True)


def _run_router(xs, p):
    return pl.pallas_call(
        _router_kernel,
        out_shape=(jax.ShapeDtypeStruct((T, D_MODEL), F32),
                   jax.ShapeDtypeStruct((T, NUM_EXPERTS), F32),
                   jax.ShapeDtypeStruct((1, 1), F32)),
    )(xs, p['norm_w'].reshape(1, D_MODEL), p['gate_w'],
      p['entropy_w'], p['entropy_b'].reshape(1, NUM_EXPERTS),
      p['temperature'].reshape(1, 1))


_CJ = 512
_NCJ = D_HID // _CJ


def _gelu_exact(v):
    return 0.5 * v * (1.0 + jax.lax.erf(v * 0.7071067811865476))


def _shift_down(h, s, width):
    if s == 0:
        return h
    return jnp.concatenate([jnp.zeros((s, width), F32), h[:T - s]], axis=0)


def _conv_experts_kernel(xn_ref, x_ref, wmask_ref, win_ref, bin_ref, kw_ref,
                         kb_ref, wout_ref, bout_ref, out_ref):
    e = pl.program_id(0)
    j = pl.program_id(1)
    xn = xn_ref[...]
    h = _mm(xn, win_ref[0]) + bin_ref[0]
    h = _gelu_exact(h)
    hc = kb_ref[0]
    for tap in range(KCONV):
        hc = hc + kw_ref[0, tap, :][None, :] * _shift_down(h, KCONV - 1 - tap,
                                                           _CJ)
    h2 = _gelu_exact(hc)
    part = _mm(h2, wout_ref[0])
    eidx = jax.lax.broadcasted_iota(jnp.int32, (1, NUM_EXPERTS), 1)
    wm = jnp.sum(wmask_ref[...] * (eidx == e).astype(F32), axis=1,
                 keepdims=True)
    contrib = part * wm
    contrib = jnp.where(j == _NCJ - 1, contrib + bout_ref[0] * wm, contrib)

    @pl.when((e == 0) & (j == 0))
    def _init():
        out_ref[...] = x_ref[...] + contrib

    @pl.when(jnp.logical_not((e == 0) & (j == 0)))
    def _acc():
        out_ref[...] = out_ref[...] + contrib


def _run_conv_experts(xn, xs, wmask, cw):
    grid = (N_CONV, _NCJ)
    return pl.pallas_call(
        _conv_experts_kernel,
        grid=grid,
        in_specs=[
            pl.BlockSpec((T, D_MODEL), lambda e, j: (0, 0)),
            pl.BlockSpec((T, D_MODEL), lambda e, j: (0, 0)),
            pl.BlockSpec((T, NUM_EXPERTS), lambda e, j: (0, 0)),
            pl.BlockSpec((1, D_MODEL, _CJ), lambda e, j: (e, 0, j)),
            pl.BlockSpec((1, 1, _CJ), lambda e, j: (e, 0, j)),
            pl.BlockSpec((1, 8, _CJ), lambda e, j: (e, 0, j)),
            pl.BlockSpec((1, 1, _CJ), lambda e, j: (e, 0, j)),
            pl.BlockSpec((1, _CJ, D_MODEL), lambda e, j: (e, j, 0)),
            pl.BlockSpec((1, 1, D_MODEL), lambda e, j: (e, 0, 0)),
        ],
        out_specs=pl.BlockSpec((T, D_MODEL), lambda e, j: (0, 0)),
        out_shape=jax.ShapeDtypeStruct((T, D_MODEL), F32),
    )(xn, xs, wmask, cw['win'], cw['bin'], cw['kw'], cw['kb'], cw['wout'],
      cw['bout'])


_MJ = 512
_NMJ = D_HID // _MJ


def _mamba_front_kernel(xn_ref, wxi_ref, wres_ref, kw_ref, kb_ref, wx_ref,
                        xi_ref, res_ref, dbc_ref):
    j = pl.program_id(1)
    xn = xn_ref[...]
    xi = _mm(xn, wxi_ref[0])
    hc = kb_ref[0]
    for tap in range(D_CONV):
        hc = hc + kw_ref[0, tap, :][None, :] * _shift_down(xi, D_CONV - 1 - tap,
                                                           _MJ)
    xi_s = jax.nn.silu(hc)
    xi_ref[0] = xi_s
    res = _mm(xn, wres_ref[0])
    res_ref[0] = jax.nn.silu(res)
    part = _mm(xi_s, wx_ref[0])

    @pl.when(j == 0)
    def _init():
        dbc_ref[0] = part

    @pl.when(j != 0)
    def _acc():
        dbc_ref[0] = dbc_ref[0] + part


def _run_mamba_front(xn, mw):
    grid = (N_MAMBA, _NMJ)
    return pl.pallas_call(
        _mamba_front_kernel,
        grid=grid,
        in_specs=[
            pl.BlockSpec((T, D_MODEL), lambda e, j: (0, 0)),
            pl.BlockSpec((1, D_MODEL, _MJ), lambda e, j: (e, 0, j)),
            pl.BlockSpec((1, D_MODEL, _MJ), lambda e, j: (e, 0, j)),
            pl.BlockSpec((1, 8, _MJ), lambda e, j: (e, 0, j)),
            pl.BlockSpec((1, 1, _MJ), lambda e, j: (e, 0, j)),
            pl.BlockSpec((1, _MJ, 128), lambda e, j: (e, j, 0)),
        ],
        out_specs=[
            pl.BlockSpec((1, T, _MJ), lambda e, j: (e, 0, j)),
            pl.BlockSpec((1, T, _MJ), lambda e, j: (e, 0, j)),
            pl.BlockSpec((1, T, 128), lambda e, j: (e, 0, 0)),
        ],
        out_shape=[
            jax.ShapeDtypeStruct((N_MAMBA, T, D_HID), F32),
            jax.ShapeDtypeStruct((N_MAMBA, T, D_HID), F32),
            jax.ShapeDtypeStruct((N_MAMBA, T, 128), F32),
        ],
    )(xn, mw['wxi'], mw['wres'], mw['kw'], mw['kb'], mw['wx'])


_SJ = 256
_NSJ = D_HID // _SJ
_NC = 64
_CL = T // _NC


def _scan_kernel(xi_ref, dbc_ref, wdt_ref, bdt_ref, at_ref, d_ref,
                 y_ref, ys_scr, srel_scr):
    dbc = dbc_ref[0]
    dt = _mm(dbc, wdt_ref[0]) + bdt_ref[0]
    delta = jnp.maximum(dt, 0.0) + jnp.log1p(jnp.exp(-jnp.abs(dt)))
    delta_r = delta.reshape(_NC, _CL, _SJ)
    xi = xi_ref[0]
    xi_r = xi.reshape(_NC, _CL, _SJ)
    dbc_r = dbc.reshape(_NC, _CL, 128)
    b_all = jnp.swapaxes(dbc_r[:, :, DT_RANK:DT_RANK + D_STATE], 1, 2)
    c_all = jnp.swapaxes(
        dbc_r[:, :, DT_RANK + D_STATE:DT_RANK + 2 * D_STATE], 1, 2)
    at = at_ref[0]

    h = jnp.zeros((_NC, D_STATE, _SJ), F32)
    ssum = jnp.zeros((_NC, 1, _SJ), F32)
    for l in range(_CL):
        d_l = delta_r[:, l:l + 1, :]
        dA = jnp.exp(d_l * at[None, :, :])
        b_l = b_all[:, :, l:l + 1]
        x_l = xi_r[:, l:l + 1, :]
        h = dA * h + (d_l * x_l) * b_l
        c_l = c_all[:, :, l:l + 1]
        ys_scr[:, l, :] = jnp.sum(h * c_l, axis=1)
        ssum = ssum + d_l
        srel_scr[:, l, :] = ssum[:, 0, :]

    P = jnp.exp(at[None, :, :] * ssum)
    H = h
    k = 1
    while k < _NC:
        Hs = jnp.concatenate([jnp.zeros((k, D_STATE, _SJ), F32), H[:_NC - k]],
                             axis=0)
        Ps = jnp.concatenate([jnp.ones((k, D_STATE, _SJ), F32), P[:_NC - k]],
                             axis=0)
        H = H + P * Hs
        P = P * Ps
        k *= 2
    hprev = jnp.concatenate([jnp.zeros((1, D_STATE, _SJ), F32), H[:_NC - 1]],
                            axis=0)

    srel = srel_scr[...]
    corr = jnp.zeros((_NC, _CL, _SJ), F32)
    for n in range(D_STATE):
        c_n = dbc_r[:, :, DT_RANK + D_STATE + n:DT_RANK + D_STATE + n + 1]
        decay = jnp.exp(at[n, :][None, None, :] * srel)
        corr = corr + c_n * decay * hprev[:, n:n + 1, :]
    y = ys_scr[...] + corr
    y_ref[0] = y.reshape(T, _SJ) + xi * d_ref[0]


def _run_scan(xi_s, dbc, mw):
    grid = (N_MAMBA, _NSJ)
    return pl.pallas_call(
        _scan_kernel,
        grid=grid,
        in_specs=[
            pl.BlockSpec((1, T, _SJ), lambda e, j: (e, 0, j)),
            pl.BlockSpec((1, T, 128), lambda e, j: (e, 0, 0)),
            pl.BlockSpec((1, 128, _SJ), lambda e, j: (e, 0, j)),
            pl.BlockSpec((1, 1, _SJ), lambda e, j: (e, 0, j)),
            pl.BlockSpec((1, D_STATE, _SJ), lambda e, j: (e, 0, j)),
            pl.BlockSpec((1, 1, _SJ), lambda e, j: (e, 0, j)),
        ],
        out_specs=pl.BlockSpec((1, T, _SJ), lambda e, j: (e, 0, j)),
        out_shape=jax.ShapeDtypeStruct((N_MAMBA, T, D_HID), F32),
        scratch_shapes=[
            pltpu.VMEM((_NC, _CL, _SJ), F32),
            pltpu.VMEM((_NC, _CL, _SJ), F32),
        ],
    )(xi_s, dbc, mw['wdt'], mw['bdt'], mw['at'], mw['d'])


def _mamba_tail_kernel(y_ref, res_ref, wmask_ref, wout_ref, base_ref, out_ref):
    e = pl.program_id(0)
    j = pl.program_id(1)
    g = y_ref[0] * res_ref[0]
    part = _mm(g, wout_ref[0])
    eidx = jax.lax.broadcasted_iota(jnp.int32, (1, NUM_EXPERTS), 1)
    wm = jnp.sum(wmask_ref[...] * (eidx == (e + N_CONV)).astype(F32), axis=1,
                 keepdims=True)
    contrib = part * wm

    @pl.when((e == 0) & (j == 0))
    def _init():
        out_ref[...] = base_ref[...] + contrib

    @pl.when(jnp.logical_not((e == 0) & (j == 0)))
    def _acc():
        out_ref[...] = out_ref[...] + contrib


def _run_mamba_tail(y_full, res_s, wmask, base, mw):
    grid = (N_MAMBA, _NMJ)
    return pl.pallas_call(
        _mamba_tail_kernel,
        grid=grid,
        in_specs=[
            pl.BlockSpec((1, T, _MJ), lambda e, j: (e, 0, j)),
            pl.BlockSpec((1, T, _MJ), lambda e, j: (e, 0, j)),
            pl.BlockSpec((T, NUM_EXPERTS), lambda e, j: (0, 0)),
            pl.BlockSpec((1, _MJ, D_MODEL), lambda e, j: (e, j, 0)),
            pl.BlockSpec((T, D_MODEL), lambda e, j: (0, 0)),
        ],
        out_specs=pl.BlockSpec((T, D_MODEL), lambda e, j: (0, 0)),
        out_shape=jax.ShapeDtypeStruct((T, D_MODEL), F32),
    )(y_full, res_s, wmask, mw['wout'], base)


def _stack_weights(params):
    ce = params['conv_experts']
    cw = {
        'win': jnp.stack([p['fc_in_w'] for p in ce]).astype(BF16),
        'bin': jnp.stack([p['fc_in_b'].reshape(1, D_HID) for p in ce]),
        'kw': jnp.stack([
            jnp.pad(jnp.transpose(p['conv_w'][:, 0, :]), ((0, 1), (0, 0)))
            for p in ce]),
        'kb': jnp.stack([p['conv_b'].reshape(1, D_HID) for p in ce]),
        'wout': jnp.stack([p['fc_out_w'] for p in ce]).astype(BF16),
        'bout': jnp.stack([p['fc_out_b'].reshape(1, D_MODEL) for p in ce]),
    }
    me = params['mamba_experts']
    mw = {
        'wxi': jnp.stack([p['in_proj_w'][:, :D_HID] for p in me]).astype(BF16),
        'wres': jnp.stack([p['in_proj_w'][:, D_HID:] for p in me]).astype(BF16),
        'kw': jnp.stack([
            jnp.pad(jnp.transpose(p['conv_w'][:, 0, :]), ((0, 4), (0, 0)))
            for p in me]),
        'kb': jnp.stack([p['conv_b'].reshape(1, D_HID) for p in me]),
        'wx': jnp.stack([
            jnp.pad(p['x_proj_w'], ((0, 0), (0, 128 - (DT_RANK + 2 * D_STATE))))
            for p in me]).astype(BF16),
        'wdt': jnp.stack([
            jnp.pad(p['dt_proj_w'], ((0, 128 - DT_RANK), (0, 0)))
            for p in me]).astype(BF16),
        'bdt': jnp.stack([p['dt_proj_b'].reshape(1, D_HID) for p in me]),
        'at': jnp.stack([jnp.transpose(-jnp.exp(p['A_log'])) for p in me]),
        'd': jnp.stack([p['D'].reshape(1, D_HID) for p in me]),
        'wout': jnp.stack([p['out_proj_w'] for p in me]).astype(BF16),
    }
    return cw, mw


@jax.jit
def kernel(x, params):
    xs = x[0]
    cw, mw = _stack_weights(params)
    xn, wmask, aux = _run_router(xs, params)
    base = _run_conv_experts(xn, xs, wmask, cw)
    xi_s, res_s, dbc = _run_mamba_front(xn, mw)
    y_full = _run_scan(xi_s, dbc, mw)
    out = _run_mamba_tail(y_full, res_s, wmask, base, mw)
    return out[None], aux[0, 0]
